# Initial kernel scaffold; baseline (speedup 1.0000x reference)
import jax
import jax.numpy as jnp
from jax.experimental import pallas as pl


def kernel(x, table):
    raise NotImplementedError("write your pallas kernel here")


# speedup vs baseline: 2.4874x; 2.4874x over previous
import functools

import jax
import jax.numpy as jnp
from jax import lax
from jax.experimental import pallas as pl
from jax.experimental.pallas import tpu as pltpu
from jax.experimental.pallas import tpu_sc as plsc


def _make_gather(n, vocab, d):
    info = plsc.get_sparse_core_info()
    nc, ns = info.num_cores, info.num_subcores
    nw = nc * ns
    assert n % nw == 0
    per_w = n // nw
    chunk = 2048
    assert per_w % chunk == 0
    n_chunks = per_w // chunk

    mesh = plsc.VectorSubcoreMesh(core_axis_name="c", subcore_axis_name="s")

    @functools.partial(
        pl.kernel,
        out_type=jax.ShapeDtypeStruct((n, d), jnp.float32),
        mesh=mesh,
        scratch_types=[
            pltpu.VMEM((chunk,), jnp.int32),
            pltpu.VMEM((chunk, d), jnp.float32),
            pltpu.SemaphoreType.DMA,
        ],
        compiler_params=pltpu.CompilerParams(use_tc_tiling_on_sc=False),
    )
    def gather_kernel(x_hbm, table_hbm, out_hbm, idx_v, rows_v, sem):
        wid = lax.axis_index("s") * nc + lax.axis_index("c")
        base = wid * per_w

        def chunk_body(i, carry):
            off = base + i * chunk
            pltpu.sync_copy(x_hbm.at[pl.ds(off, chunk)], idx_v)
            pltpu.async_copy(table_hbm.at[idx_v], rows_v, sem).wait()
            pltpu.sync_copy(rows_v, out_hbm.at[pl.ds(off, chunk)])
            return carry

        lax.fori_loop(0, n_chunks, chunk_body, 0)

    return gather_kernel


def kernel(x, table):
    r, c = x.shape
    vocab, d = table.shape
    n = r * c
    xf = x.reshape(n)
    out = _make_gather(n, vocab, d)(xf, table)
    return out.reshape(r, c, d)


# speedup vs baseline: 2.5320x; 1.0179x over previous
import functools

import jax
import jax.numpy as jnp
from jax import lax
from jax.experimental import pallas as pl
from jax.experimental.pallas import tpu as pltpu
from jax.experimental.pallas import tpu_sc as plsc

_NBUF = 2


def _make_gather(n, vocab, d, chunk):
    info = plsc.get_sparse_core_info()
    nc, ns = info.num_cores, info.num_subcores
    nw = nc * ns
    assert n % nw == 0
    per_w = n // nw
    assert per_w % chunk == 0
    n_chunks = per_w // chunk
    assert n_chunks % _NBUF == 0 and n_chunks >= 2 * _NBUF

    mesh = plsc.VectorSubcoreMesh(core_axis_name="c", subcore_axis_name="s")

    @functools.partial(
        pl.kernel,
        out_type=jax.ShapeDtypeStruct((n, d), jnp.float32),
        mesh=mesh,
        scratch_types=[
            pltpu.VMEM((chunk,), jnp.int32),
            pltpu.VMEM((chunk,), jnp.int32),
            pltpu.VMEM((chunk, d), jnp.float32),
            pltpu.VMEM((chunk, d), jnp.float32),
            pltpu.SemaphoreType.DMA,
            pltpu.SemaphoreType.DMA,
            pltpu.SemaphoreType.DMA,
        ],
        compiler_params=pltpu.CompilerParams(use_tc_tiling_on_sc=False),
    )
    def gather_kernel(x_hbm, table_hbm, out_hbm, idx_v0, idx_v1, rows_v0,
                      rows_v1, idx_sem, gat_sem, out_sem):
        wid = lax.axis_index("s") * nc + lax.axis_index("c")
        base = wid * per_w
        idx_bufs = (idx_v0, idx_v1)
        row_bufs = (rows_v0, rows_v1)

        def idx_start(i, b):
            pltpu.async_copy(x_hbm.at[pl.ds(base + i * chunk, chunk)],
                             idx_bufs[b], idx_sem)

        def idx_wait(b):
            pltpu.make_async_copy(x_hbm.at[pl.ds(base, chunk)],
                                  idx_bufs[b], idx_sem).wait()

        def gat_start(b):
            pltpu.async_copy(table_hbm.at[idx_bufs[b]], row_bufs[b], gat_sem)

        def gat_wait(b):
            pltpu.make_async_copy(table_hbm.at[idx_bufs[b]],
                                  row_bufs[b], gat_sem).wait()

        def out_start(i, b):
            pltpu.async_copy(row_bufs[b],
                             out_hbm.at[pl.ds(base + i * chunk, chunk)], out_sem)

        def out_wait(b):
            pltpu.make_async_copy(row_bufs[b],
                                  out_hbm.at[pl.ds(base, chunk)], out_sem).wait()

        idx_start(0, 0)
        idx_start(1, 1)
        for b in range(_NBUF):
            idx_wait(b)
            gat_start(b)
            gat_wait(b)
            idx_start(2 + b, b)
            out_start(b, b)

        def pair_body(io, carry):
            for b in range(_NBUF):
                i = io * _NBUF + b
                idx_wait(b)
                out_wait(b)
                gat_start(b)
                gat_wait(b)
                idx_start(i + 2, b)
                out_start(i, b)
            return carry

        lax.fori_loop(1, n_chunks // _NBUF - 1, pair_body, 0)

        for b in range(_NBUF):
            i = n_chunks - _NBUF + b
            idx_wait(b)
            out_wait(b)
            gat_start(b)
            gat_wait(b)
            out_start(i, b)
        for b in range(_NBUF):
            out_wait(b)

    return gather_kernel


def kernel(x, table):
    r, c = x.shape
    vocab, d = table.shape
    n = r * c
    xf = x.reshape(n)
    out = _make_gather(n, vocab, d, 2048)(xf, table)
    return out.reshape(r, c, d)
